# Initial kernel scaffold; baseline (speedup 1.0000x reference)
import jax
import jax.numpy as jnp
from jax.experimental import pallas as pl


def kernel(self_vecs, neigh_vecs, self_weights, neigh_weights, attention_weights):
    raise NotImplementedError("write your pallas kernel here")


# speedup vs baseline: 4.9567x; 4.9567x over previous
import jax
import jax.numpy as jnp
from jax.experimental import pallas as pl
from jax.experimental.pallas import tpu as pltpu

_N = 10000
_K = 32
_D = 128
_NS = 16
_BN = 400


def _fused_body(self_ref, neigh_ref, sw_ref, nw_ref, att_ref, out_ref):
    sv = self_ref[...]
    nb = neigh_ref[...]
    sw = sw_ref[...]
    nw = nw_ref[...]
    att = att_ref[...]

    f32 = jnp.float32
    u = jax.lax.dot_general(sw, att, (((1,), (1,)), ((), ())),
                            preferred_element_type=f32)
    v = jax.lax.dot_general(nw, att, (((1,), (1,)), ((), ())),
                            preferred_element_type=f32)

    self_logit = jax.nn.relu(
        jax.lax.dot_general(sv, u, (((1,), (0,)), ((), ())),
                            preferred_element_type=f32))
    nl = jax.lax.dot_general(nb.reshape(_BN * _K, _D), v,
                             (((1,), (0,)), ((), ())),
                             preferred_element_type=f32)
    neigh_logits = jax.nn.relu(nl.reshape(_BN, _K))

    lt = neigh_logits.T
    st = self_logit.T

    m = jnp.maximum(jnp.max(lt, axis=0, keepdims=True), st)
    en = jnp.exp(lt - m)
    es = jnp.exp(st - m)
    z = es + jnp.sum(en, axis=0, keepdims=True)
    s = en / z

    iota_k = jax.lax.broadcasted_iota(jnp.int32, (_K, _BN), 0)
    rank = jnp.zeros((_K, _BN), dtype=jnp.int32)
    for j in range(_K):
        row = s[j:j + 1, :]
        gt = row > s
        eq_lower = jnp.logical_and(row == s, iota_k > j)
        rank = rank + jnp.logical_or(gt, eq_lower).astype(jnp.int32)
    w = jnp.where(rank < _NS, s, 0.0)

    wt = w.T
    combined = jnp.sum(wt[:, :, None] * nb, axis=1)
    st_out = jax.lax.dot_general(sv, sw, (((1,), (0,)), ((), ())),
                                 preferred_element_type=f32)
    cn = jax.lax.dot_general(combined, nw, (((1,), (0,)), ((), ())),
                             preferred_element_type=f32)
    out_ref[...] = jax.nn.relu(st_out + cn)


def kernel(self_vecs, neigh_vecs, self_weights, neigh_weights, attention_weights):
    att = attention_weights.reshape(1, _D)
    grid = (_N // _BN,)
    return pl.pallas_call(
        _fused_body,
        grid=grid,
        in_specs=[
            pl.BlockSpec((_BN, _D), lambda i: (i, 0)),
            pl.BlockSpec((_BN, _K, _D), lambda i: (i, 0, 0)),
            pl.BlockSpec((_D, _D), lambda i: (0, 0)),
            pl.BlockSpec((_D, _D), lambda i: (0, 0)),
            pl.BlockSpec((1, _D), lambda i: (0, 0)),
        ],
        out_specs=pl.BlockSpec((_BN, _D), lambda i: (i, 0)),
        out_shape=jax.ShapeDtypeStruct((_N, _D), jnp.float32),
        compiler_params=pltpu.CompilerParams(
            dimension_semantics=("arbitrary",),
        ),
    )(self_vecs, neigh_vecs, self_weights, neigh_weights, att)


# speedup vs baseline: 5.3978x; 1.0890x over previous
import jax
import jax.numpy as jnp
from jax.experimental import pallas as pl
from jax.experimental.pallas import tpu as pltpu

_N = 10000
_K = 32
_D = 128
_NS = 16
_BN = 1000


def _fused_body(self_ref, neigh_ref, sw_ref, nw_ref, att_ref, out_ref):
    sv = self_ref[...]
    nb = neigh_ref[...]
    sw = sw_ref[...]
    nw = nw_ref[...]
    att = att_ref[...]

    f32 = jnp.float32
    u = jax.lax.dot_general(sw, att, (((1,), (1,)), ((), ())),
                            preferred_element_type=f32)
    v = jax.lax.dot_general(nw, att, (((1,), (1,)), ((), ())),
                            preferred_element_type=f32)

    self_logit = jax.nn.relu(
        jax.lax.dot_general(sv, u, (((1,), (0,)), ((), ())),
                            preferred_element_type=f32))
    nl = jax.lax.dot_general(nb.reshape(_BN * _K, _D), v,
                             (((1,), (0,)), ((), ())),
                             preferred_element_type=f32)
    neigh_logits = jax.nn.relu(nl.reshape(_BN, _K))

    lt = neigh_logits.T
    st = self_logit.T

    m = jnp.maximum(jnp.max(lt, axis=0, keepdims=True), st)
    en = jnp.exp(lt - m)
    es = jnp.exp(st - m)
    z = es + jnp.sum(en, axis=0, keepdims=True)
    s = en / z

    iota_k = jax.lax.broadcasted_iota(jnp.int32, (_K, _BN), 0)
    key = jnp.bitwise_or(
        jnp.bitwise_and(jax.lax.bitcast_convert_type(s, jnp.int32), ~31),
        (_K - 1) - iota_k)
    rank = jnp.zeros((_K, _BN), dtype=jnp.int32)
    for j in range(_K):
        row = key[j:j + 1, :]
        rank = rank + (row > key).astype(jnp.int32)
    w = jnp.where(rank < _NS, s, 0.0)

    wt = w.T
    combined = jnp.sum(wt[:, :, None] * nb, axis=1)
    st_out = jax.lax.dot_general(sv, sw, (((1,), (0,)), ((), ())),
                                 preferred_element_type=f32)
    cn = jax.lax.dot_general(combined, nw, (((1,), (0,)), ((), ())),
                             preferred_element_type=f32)
    out_ref[...] = jax.nn.relu(st_out + cn)


def kernel(self_vecs, neigh_vecs, self_weights, neigh_weights, attention_weights):
    att = attention_weights.reshape(1, _D)
    grid = (_N // _BN,)
    return pl.pallas_call(
        _fused_body,
        grid=grid,
        in_specs=[
            pl.BlockSpec((_BN, _D), lambda i: (i, 0)),
            pl.BlockSpec((_BN, _K, _D), lambda i: (i, 0, 0)),
            pl.BlockSpec((_D, _D), lambda i: (0, 0)),
            pl.BlockSpec((_D, _D), lambda i: (0, 0)),
            pl.BlockSpec((1, _D), lambda i: (0, 0)),
        ],
        out_specs=pl.BlockSpec((_BN, _D), lambda i: (i, 0)),
        out_shape=jax.ShapeDtypeStruct((_N, _D), jnp.float32),
        compiler_params=pltpu.CompilerParams(
            dimension_semantics=("arbitrary",),
        ),
    )(self_vecs, neigh_vecs, self_weights, neigh_weights, att)
